# baseline (speedup 1.0000x reference)
import functools

import jax
import jax.numpy as jnp
from jax import lax
from jax.experimental import pallas as pl
from jax.experimental.pallas import tpu as pltpu
from jax.experimental.pallas import tpu_sc as plsc


def _gather_to_physical(ids_p, table, S, BT):
    info = plsc.get_sparse_core_info()
    nw = info.num_cores * info.num_subcores
    assert BT == nw

    mesh = plsc.VectorSubcoreMesh(core_axis_name="c", subcore_axis_name="s")
    nblk = S // 2

    @functools.partial(
        pl.kernel,
        mesh=mesh,
        out_type=jax.ShapeDtypeStruct((S, 8, BT, 8, 128), jnp.float32),
        compiler_params=pltpu.CompilerParams(use_tc_tiling_on_sc=False,
                                             needs_layout_passes=False),
        scratch_types=(
            [pltpu.VMEM((S // 8, 8, 128), jnp.int32)]
            + [pltpu.VMEM((256,), jnp.int32)] * 3
            + [pltpu.VMEM((256, 64), jnp.float32)] * 3
            + [pltpu.VMEM((16, 8, 130), jnp.float32)] * 3
            + [pltpu.SemaphoreType.DMA] * 6
        ),
    )
    def k(ids_hbm, table_hbm, out_hbm, idx_all,
          i0, i1, i2, r0, r1, r2, t0, t1, t2, g0, g1, g2, w0, w1, w2):
        wid = lax.axis_index("s") * info.num_cores + lax.axis_index("c")
        istage = (i0, i1, i2)
        rows = (r0, r1, r2)
        tbuf = (t0, t1, t2)
        gsem = (g0, g1, g2)
        wsem = (w0, w1, w2)

        pltpu.sync_copy(ids_hbm.at[:, wid], idx_all)

        iota = lax.iota(jnp.int32, 16)
        zeros = jnp.zeros((16,), jnp.int32)
        cvecs = [((c0 + iota) // 8, (c0 + iota) % 8) for c0 in (0, 16, 32, 48)]

        def start_gather(i, slot):
            st = istage[slot]
            for sl in range(2):
                s = 2 * i + sl
                for j in range(8):
                    st[pl.ds(128 * sl + 16 * j, 16)] = (
                        idx_all[s // 8, s % 8, pl.ds(16 * j, 16)] * 2)
            pltpu.async_copy(table_hbm.at[st], rows[slot], gsem[slot])

        def wait_gather(slot):
            pltpu.make_async_copy(table_hbm.at[istage[slot]], rows[slot],
                                  gsem[slot]).wait()

        def transpose(slot):
            r = rows[slot]
            t = tbuf[slot]

            @plsc.parallel_loop(0, 256, unroll=4)
            def _(b):
                bvec = zeros + (b & 127)
                sg = (b >> 7) * 8
                for q, (cgv, civ) in enumerate(cvecs):
                    v = r[b, pl.ds(16 * q, 16)]
                    plsc.store_scatter(t, [cgv + sg, civ, bvec], v)

        def start_write(i, slot):
            for sl in range(2):
                pltpu.async_copy(
                    tbuf[slot].at[pl.ds(8 * sl, 8), :, pl.ds(0, 128)],
                    out_hbm.at[2 * i + sl, :, wid], wsem[slot])

        def wait_write(i, slot):
            for sl in range(2):
                pltpu.make_async_copy(
                    tbuf[slot].at[pl.ds(8 * sl, 8), :, pl.ds(0, 128)],
                    out_hbm.at[2 * i + sl, :, wid], wsem[slot]).wait()

        nb = 3
        main_iters = (nblk - nb) // nb
        rem_start = main_iters * nb
        for q in range(nb):
            start_gather(q, q)

        def body(p, carry):
            i = nb * p
            for q in range(nb):
                @pl.when(p > 0)
                def _(q=q):
                    wait_write(i - nb + q, q)

                wait_gather(q)
                transpose(q)
                start_write(i + q, q)
                start_gather(i + nb + q, q)
            return carry

        lax.fori_loop(0, main_iters, body, 0)

        for i in range(rem_start, nblk):
            slot = i % nb
            wait_write(i - nb, slot)
            wait_gather(slot)
            transpose(slot)
            start_write(i, slot)
            if i + nb < nblk:
                start_gather(i + nb, slot)

        for i in range(nblk - nb, nblk):
            wait_write(i, i % nb)

    return k(ids_p, table)


def kernel(token_ids, table):
    b0, s = token_ids.shape
    v, d = table.shape
    bt = b0 // 128
    ids_p = token_ids.astype(jnp.int32).reshape(bt, 128, s // 8, 8)
    ids_p = ids_p.transpose(2, 0, 3, 1)
    table_p = jnp.pad(table, ((0, 0), (0, d))).reshape(2 * v, d)
    out_p = _gather_to_physical(ids_p, table_p, s, bt)
    out = out_p.reshape(s, 8, bt, 8, 128).transpose(2, 4, 0, 1, 3)
    return out.reshape(b0, s, d)


# speedup vs baseline: 1.0076x; 1.0076x over previous
import functools

import jax
import jax.numpy as jnp
from jax import lax
from jax.experimental import pallas as pl
from jax.experimental.pallas import tpu as pltpu
from jax.experimental.pallas import tpu_sc as plsc


def _gather_to_physical(ids_p, table, S, BT):
    info = plsc.get_sparse_core_info()
    nw = info.num_cores * info.num_subcores
    assert BT == nw

    mesh = plsc.VectorSubcoreMesh(core_axis_name="c", subcore_axis_name="s")

    @functools.partial(
        pl.kernel,
        mesh=mesh,
        out_type=jax.ShapeDtypeStruct((S, 8, BT, 8, 128), jnp.float32),
        compiler_params=pltpu.CompilerParams(use_tc_tiling_on_sc=False,
                                             needs_layout_passes=False),
        scratch_types=(
            [pltpu.VMEM((S // 8, 8, 128), jnp.int32)]
            + [pltpu.VMEM((128,), jnp.int32)] * 4
            + [pltpu.VMEM((128, 64), jnp.float32)] * 4
            + [pltpu.VMEM((8, 8, 130), jnp.float32)] * 4
            + [pltpu.SemaphoreType.DMA] * 8
        ),
    )
    def k(ids_hbm, table_hbm, out_hbm, idx_all,
          i0, i1, i2, i3, r0, r1, r2, r3, t0, t1, t2, t3,
          g0, g1, g2, g3, w0, w1, w2, w3):
        wid = lax.axis_index("s") * info.num_cores + lax.axis_index("c")
        istage = (i0, i1, i2, i3)
        rows = (r0, r1, r2, r3)
        tbuf = (t0, t1, t2, t3)
        gsem = (g0, g1, g2, g3)
        wsem = (w0, w1, w2, w3)

        pltpu.sync_copy(ids_hbm.at[:, wid], idx_all)

        iota = lax.iota(jnp.int32, 16)
        zeros = jnp.zeros((16,), jnp.int32)
        cvecs = [((c0 + iota) // 8, (c0 + iota) % 8) for c0 in (0, 16, 32, 48)]

        def start_gather(s, slot):
            st = istage[slot]
            for j in range(8):
                st[pl.ds(16 * j, 16)] = (
                    idx_all[s // 8, s % 8, pl.ds(16 * j, 16)] * 2)
            pltpu.async_copy(table_hbm.at[st], rows[slot], gsem[slot])

        def wait_gather(slot):
            pltpu.make_async_copy(table_hbm.at[istage[slot]], rows[slot],
                                  gsem[slot]).wait()

        def transpose(slot):
            r = rows[slot]
            t = tbuf[slot]

            @plsc.parallel_loop(0, 128, unroll=4)
            def _(b):
                bvec = zeros + b
                for q, (cgv, civ) in enumerate(cvecs):
                    v = r[b, pl.ds(16 * q, 16)]
                    plsc.store_scatter(t, [cgv, civ, bvec], v)

        def start_write(s, slot):
            pltpu.async_copy(tbuf[slot].at[:, :, pl.ds(0, 128)],
                             out_hbm.at[s, :, wid], wsem[slot])

        def wait_write(s, slot):
            pltpu.make_async_copy(tbuf[slot].at[:, :, pl.ds(0, 128)],
                                  out_hbm.at[s, :, wid],
                                  wsem[slot]).wait()

        nb = 4
        for q in range(nb):
            start_gather(q, q)

        def body(p, carry):
            s = nb * p
            for q in range(nb):
                @pl.when(p > 0)
                def _(q=q):
                    wait_write(s - nb + q, q)

                wait_gather(q)
                transpose(q)
                start_write(s + q, q)
                start_gather(s + nb + q, q)
            return carry

        lax.fori_loop(0, S // nb - 1, body, 0)

        for q in range(nb):
            wait_write(S - 2 * nb + q, q)
            wait_gather(q)
            transpose(q)
            start_write(S - nb + q, q)

        for q in range(nb):
            wait_write(S - nb + q, q)

    return k(ids_p, table)


def kernel(token_ids, table):
    b0, s = token_ids.shape
    v, d = table.shape
    bt = b0 // 128
    ids_p = token_ids.astype(jnp.int32).reshape(bt, 128, s // 8, 8)
    ids_p = ids_p.transpose(2, 0, 3, 1)
    table_p = jnp.pad(table, ((0, 0), (0, d))).reshape(2 * v, d)
    out_p = _gather_to_physical(ids_p, table_p, s, bt)
    out = out_p.reshape(s, 8, bt, 8, 128).transpose(2, 4, 0, 1, 3)
    return out.reshape(b0, s, d)
